# baseline (speedup 1.0000x reference)
import functools

import jax
import jax.numpy as jnp
from jax import lax
from jax.experimental import pallas as pl
from jax.experimental.pallas import tpu as pltpu
from jax.experimental.pallas import tpu_sc as plsc

B, T, V, D = 4, 2048, 100000, 2048

_info = plsc.get_sparse_core_info()
NC, NS = _info.num_cores, _info.num_subcores
NW = NC * NS

N_TOK = B * T
ROWS_PER_W = N_TOK // NW
CHUNK = 16
NCH = ROWS_PER_W // CHUNK
NB = 3


def _gather_body(table_hbm, idx_hbm, out_hbm, idx_v, bufs, gsems, osems):
    c = lax.axis_index("c")
    s = lax.axis_index("s")
    wid = s * NC + c
    pltpu.sync_copy(idx_hbm.at[wid], idx_v)
    base = wid * ROWS_PER_W

    def out_slice(ch):
        return out_hbm.at[pl.ds(base + ch * CHUNK, CHUNK)]

    g, o = {}, {}
    g[0] = pltpu.async_copy(table_hbm.at[idx_v.at[0]], bufs.at[0], gsems.at[0])
    for ch in range(NCH):
        b = ch % NB
        g[ch].wait()
        o[ch] = pltpu.async_copy(bufs.at[b], out_slice(ch), osems.at[b])
        nx = ch + 1
        if nx < NCH:
            nb_ = nx % NB
            if nx >= NB:
                o[nx - NB].wait()
            g[nx] = pltpu.async_copy(
                table_hbm.at[idx_v.at[nx]], bufs.at[nb_], gsems.at[nb_]
            )
    for ch in range(max(0, NCH - NB), NCH):
        o[ch].wait()


_gather = pl.kernel(
    _gather_body,
    mesh=plsc.VectorSubcoreMesh(core_axis_name="c", subcore_axis_name="s"),
    compiler_params=pltpu.CompilerParams(use_tc_tiling_on_sc=False),
    out_type=jax.ShapeDtypeStruct((N_TOK, D), jnp.float32),
    scratch_types=[
        pltpu.VMEM((NCH, CHUNK), jnp.int32),
        pltpu.VMEM((NB, CHUNK, D), jnp.float32),
        pltpu.SemaphoreType.DMA((NB,)),
        pltpu.SemaphoreType.DMA((NB,)),
    ],
)


TR = 256


def _mask_body(amask_ref, out_ref):
    r0 = pl.program_id(1) * TR
    rows = lax.broadcasted_iota(jnp.int32, (TR, T), 0) + r0
    cols = lax.broadcasted_iota(jnp.int32, (TR, T), 1)
    val = jnp.where(cols > rows, -jnp.inf, 0.0).astype(jnp.float32)
    pad = (amask_ref[0, 0, :] == 0)[None, :]
    out_ref[0, 0, :, :] = jnp.where(pad, -jnp.inf, val)


_mask = pl.pallas_call(
    _mask_body,
    grid=(B, T // TR),
    in_specs=[pl.BlockSpec((1, 1, T), lambda b, r: (b, 0, 0))],
    out_specs=pl.BlockSpec((1, 1, TR, T), lambda b, r: (b, 0, r, 0)),
    out_shape=jax.ShapeDtypeStruct((B, 1, T, T), jnp.float32),
)


@jax.jit
def kernel(input_ids, attention_mask, embed_table):
    ids = input_ids.astype(jnp.int32).reshape(NW, NCH, CHUNK)
    hidden = _gather(embed_table, ids).reshape(B, T, D)
    attn_4d = _mask(attention_mask.astype(jnp.int32).reshape(B, 1, T))
    base_pos = jnp.broadcast_to(jnp.arange(T, dtype=jnp.int32)[None, :], (B, T))
    position_ids = jnp.stack([base_pos, base_pos, base_pos], axis=0)
    return hidden, attn_4d, position_ids


# speedup vs baseline: 8.0169x; 8.0169x over previous
import functools

import jax
import jax.numpy as jnp
from jax import lax
from jax.experimental import pallas as pl
from jax.experimental.pallas import tpu as pltpu
from jax.experimental.pallas import tpu_sc as plsc

B, T, V, D = 4, 2048, 100000, 2048

_info = plsc.get_sparse_core_info()
NC, NS = _info.num_cores, _info.num_subcores
NW = NC * NS

N_TOK = B * T
ROWS_PER_W = N_TOK // NW
CHUNK = 8
NCH = ROWS_PER_W // CHUNK
NB = 6


def _gather_body(table_hbm, idx_hbm, out_hbm, idx_v, bufs, gsems, osems):
    c = lax.axis_index("c")
    s = lax.axis_index("s")
    wid = s * NC + c
    pltpu.sync_copy(idx_hbm.at[wid], idx_v)
    base = wid * ROWS_PER_W

    def out_slice(ch):
        return out_hbm.at[pl.ds(base + ch * CHUNK, CHUNK)]

    g, o = {}, {}
    g[0] = pltpu.async_copy(table_hbm.at[idx_v.at[0]], bufs.at[0], gsems.at[0])
    for ch in range(NCH):
        b = ch % NB
        g[ch].wait()
        o[ch] = pltpu.async_copy(bufs.at[b], out_slice(ch), osems.at[b])
        nx = ch + 1
        if nx < NCH:
            nb_ = nx % NB
            if nx >= NB:
                o[nx - NB].wait()
            g[nx] = pltpu.async_copy(
                table_hbm.at[idx_v.at[nx]], bufs.at[nb_], gsems.at[nb_]
            )
    for ch in range(max(0, NCH - NB), NCH):
        o[ch].wait()


_gather = pl.kernel(
    _gather_body,
    mesh=plsc.VectorSubcoreMesh(core_axis_name="c", subcore_axis_name="s"),
    out_type=jax.ShapeDtypeStruct((N_TOK, D), jnp.float32),
    scratch_types=[
        pltpu.VMEM((NCH, CHUNK), jnp.int32),
        pltpu.VMEM((NB, CHUNK, D), jnp.float32),
        pltpu.SemaphoreType.DMA((NB,)),
        pltpu.SemaphoreType.DMA((NB,)),
    ],
)


TR = 256


def _mask_body(amask_ref, out_ref):
    r0 = pl.program_id(1) * TR
    rows = lax.broadcasted_iota(jnp.int32, (TR, T), 0) + r0
    cols = lax.broadcasted_iota(jnp.int32, (TR, T), 1)
    val = jnp.where(cols > rows, -jnp.inf, 0.0).astype(jnp.float32)
    pad = (amask_ref[0, 0, :] == 0)[None, :]
    out_ref[0, 0, :, :] = jnp.where(pad, -jnp.inf, val)


_mask = pl.pallas_call(
    _mask_body,
    grid=(B, T // TR),
    in_specs=[pl.BlockSpec((1, 1, T), lambda b, r: (b, 0, 0))],
    out_specs=pl.BlockSpec((1, 1, TR, T), lambda b, r: (b, 0, r, 0)),
    out_shape=jax.ShapeDtypeStruct((B, 1, T, T), jnp.float32),
)


@jax.jit
def kernel(input_ids, attention_mask, embed_table):
    ids = input_ids.astype(jnp.int32).reshape(NW, NCH, CHUNK)
    hidden = _gather(embed_table, ids).reshape(B, T, D)
    attn_4d = _mask(attention_mask.astype(jnp.int32).reshape(B, 1, T))
    base_pos = jnp.broadcast_to(jnp.arange(T, dtype=jnp.int32)[None, :], (B, T))
    position_ids = jnp.stack([base_pos, base_pos, base_pos], axis=0)
    return hidden, attn_4d, position_ids


# speedup vs baseline: 8.2150x; 1.0247x over previous
import functools

import jax
import jax.numpy as jnp
from jax import lax
from jax.experimental import pallas as pl
from jax.experimental.pallas import tpu as pltpu
from jax.experimental.pallas import tpu_sc as plsc

B, T, V, D = 4, 2048, 100000, 2048

_info = plsc.get_sparse_core_info()
NC, NS = _info.num_cores, _info.num_subcores
NW = NC * NS

N_TOK = B * T
ROWS_PER_W = N_TOK // NW
CHUNK = 16
NCH = ROWS_PER_W // CHUNK
NB = 3


def _gather_body(table_hbm, idx_hbm, out_hbm, idx_v, bufs, gsems, osems):
    c = lax.axis_index("c")
    s = lax.axis_index("s")
    wid = s * NC + c
    pltpu.sync_copy(idx_hbm.at[wid], idx_v)
    base = wid * ROWS_PER_W

    def out_slice(ch):
        return out_hbm.at[pl.ds(base + ch * CHUNK, CHUNK)]

    g, o = {}, {}
    g[0] = pltpu.async_copy(table_hbm.at[idx_v.at[0]], bufs.at[0], gsems.at[0])
    for ch in range(NCH):
        b = ch % NB
        g[ch].wait()
        o[ch] = pltpu.async_copy(bufs.at[b], out_slice(ch), osems.at[b])
        nx = ch + 1
        if nx < NCH:
            nb_ = nx % NB
            if nx >= NB:
                o[nx - NB].wait()
            g[nx] = pltpu.async_copy(
                table_hbm.at[idx_v.at[nx]], bufs.at[nb_], gsems.at[nb_]
            )
    for ch in range(max(0, NCH - NB), NCH):
        o[ch].wait()


_gather = pl.kernel(
    _gather_body,
    mesh=plsc.VectorSubcoreMesh(core_axis_name="c", subcore_axis_name="s"),
    out_type=jax.ShapeDtypeStruct((N_TOK, D), jnp.float32),
    scratch_types=[
        pltpu.VMEM((NCH, CHUNK), jnp.int32),
        pltpu.VMEM((NB, CHUNK, D), jnp.float32),
        pltpu.SemaphoreType.DMA((NB,)),
        pltpu.SemaphoreType.DMA((NB,)),
    ],
)


TR = 256


def _mask_body(amask_ref, out_ref):
    r0 = pl.program_id(1) * TR
    rows = lax.broadcasted_iota(jnp.int32, (TR, T), 0) + r0
    cols = lax.broadcasted_iota(jnp.int32, (TR, T), 1)
    val = jnp.where(cols > rows, -jnp.inf, 0.0).astype(jnp.float32)
    pad = (amask_ref[0, 0, :] == 0)[None, :]
    out_ref[0, 0, :, :] = jnp.where(pad, -jnp.inf, val)


_mask = pl.pallas_call(
    _mask_body,
    grid=(B, T // TR),
    in_specs=[pl.BlockSpec((1, 1, T), lambda b, r: (b, 0, 0))],
    out_specs=pl.BlockSpec((1, 1, TR, T), lambda b, r: (b, 0, r, 0)),
    out_shape=jax.ShapeDtypeStruct((B, 1, T, T), jnp.float32),
)


@jax.jit
def kernel(input_ids, attention_mask, embed_table):
    ids = input_ids.astype(jnp.int32).reshape(NW, NCH, CHUNK)
    attn_4d = _mask(attention_mask.astype(jnp.int32).reshape(B, 1, T))
    hidden = _gather(embed_table, ids).reshape(B, T, D)
    base_pos = jnp.broadcast_to(jnp.arange(T, dtype=jnp.int32)[None, :], (B, T))
    position_ids = jnp.stack([base_pos, base_pos, base_pos], axis=0)
    return hidden, attn_4d, position_ids
